# baseline (speedup 1.0000x reference)
import jax
import jax.numpy as jnp
from jax import lax
from jax.experimental import pallas as pl
from jax.experimental.pallas import tpu as pltpu
from jax.experimental.pallas import tpu_sc as plsc

NUM_TIMESTEPS = 1000
BATCH = 16384
L = 16
NC, NS = 2, 16
NW = NC * NS
BPW = BATCH // NW
NCHUNK = 63
W_PACK = NCHUNK * L
TAB_PAD = 1024


def _snr_body(pack_hbm, ts_hbm, out_hbm,
              w_v, tab_v, offs_v, tmp_v, ts_v, out_v, sem_ts, sem_w):
    wid = lax.axis_index("s") * NC + lax.axis_index("c")
    base = wid * BPW

    cp_w = pltpu.async_copy(pack_hbm, w_v, sem_w)
    cp_ts = pltpu.async_copy(ts_hbm.at[pl.ds(base, BPW)], ts_v, sem_ts)

    neg_inf = jnp.full((L,), -jnp.inf, jnp.float32)
    tab_v[pl.ds(TAB_PAD - L, L)] = neg_inf

    cp_w.wait()
    zeros = jnp.zeros((L,), jnp.int32)
    wini_vec = plsc.load_gather(w_v, [zeros + NUM_TIMESTEPS])
    slope_vec = plsc.load_gather(w_v, [zeros + (NUM_TIMESTEPS + 1)])

    for j in range(NCHUNK):
        v = jnp.maximum(w_v[pl.ds(j * L, L)] + wini_vec, 0.0)
        tab_v[pl.ds(j * L, L)] = plsc.cumsum(v)

    iota = lax.iota(jnp.int32, L)
    carry = 0.0 - slope_vec
    for k in range(4):
        ends = plsc.load_gather(tab_v, [iota * L + (k * L * L + L - 1)])
        inc = plsc.cumsum(ends) + carry
        exc = inc - ends
        if k < 3:
            offs_v[pl.ds(k * L, L)] = exc
            tmp_v[...] = inc
            carry = plsc.load_gather(tmp_v, [zeros + (L - 1)])
        else:
            offs_v[pl.ds(k * L, L)] = jnp.where(iota == L - 1, 0.0, exc)

    cp_ts.wait()
    for i in range(BPW // L):
        t = ts_v[pl.ds(i * L, L)]
        idx = jnp.bitwise_and(t - 1, TAB_PAD - 1)
        out_v[pl.ds(i * L, L)] = (
            plsc.load_gather(tab_v, [idx])
            + plsc.load_gather(offs_v, [jnp.right_shift(idx, 4)]))

    pltpu.sync_copy(out_v, out_hbm.at[pl.ds(base, BPW)])


@jax.jit
def kernel(W, slope, power, w_ini, timesteps):
    del power
    pack = jnp.concatenate([
        W.astype(jnp.float32),
        jnp.reshape(w_ini.astype(jnp.float32), (1,)),
        slope.astype(jnp.float32),
        jnp.zeros((W_PACK - NUM_TIMESTEPS - 2,), jnp.float32),
    ])
    run = pl.kernel(
        _snr_body,
        out_type=jax.ShapeDtypeStruct((BATCH,), jnp.float32),
        mesh=plsc.VectorSubcoreMesh(core_axis_name="c", subcore_axis_name="s"),
        compiler_params=pltpu.CompilerParams(needs_layout_passes=False),
        scratch_types=[
            pltpu.VMEM((W_PACK,), jnp.float32),
            pltpu.VMEM((TAB_PAD,), jnp.float32),
            pltpu.VMEM((4 * L,), jnp.float32),
            pltpu.VMEM((L,), jnp.float32),
            pltpu.VMEM((BPW,), jnp.int32),
            pltpu.VMEM((BPW,), jnp.float32),
            pltpu.SemaphoreType.DMA,
            pltpu.SemaphoreType.DMA,
        ],
    )
    return run(pack, timesteps)


# speedup vs baseline: 1.0403x; 1.0403x over previous
import jax
import jax.numpy as jnp
from jax import lax
from jax.experimental import pallas as pl
from jax.experimental.pallas import tpu as pltpu
from jax.experimental.pallas import tpu_sc as plsc

NUM_TIMESTEPS = 1000
BATCH = 16384
L = 16
NC, NS = 2, 16
NW = NC * NS
BPW = BATCH // NW
NCHUNK = 63
W_PACK = NCHUNK * L
TAB_PAD = 1024


def _snr_body(pack_hbm, ts_hbm, out_hbm,
              w_v, tab_v, offs_v, tmp_v, ts_v, out_v, sem_ts, sem_w):
    wid = lax.axis_index("s") * NC + lax.axis_index("c")
    base = wid * BPW

    cp_w = pltpu.async_copy(pack_hbm.at[pl.ds(wid * W_PACK, W_PACK)], w_v,
                            sem_w)
    cp_ts = pltpu.async_copy(ts_hbm.at[pl.ds(base, BPW)], ts_v, sem_ts)

    neg_inf = jnp.full((L,), -jnp.inf, jnp.float32)
    tab_v[pl.ds(TAB_PAD - L, L)] = neg_inf

    cp_w.wait()
    zeros = jnp.zeros((L,), jnp.int32)
    wini_vec = plsc.load_gather(w_v, [zeros + NUM_TIMESTEPS])
    slope_vec = plsc.load_gather(w_v, [zeros + (NUM_TIMESTEPS + 1)])

    for j in range(NCHUNK):
        v = jnp.maximum(w_v[pl.ds(j * L, L)] + wini_vec, 0.0)
        tab_v[pl.ds(j * L, L)] = plsc.cumsum(v)

    iota = lax.iota(jnp.int32, L)
    carry = 0.0 - slope_vec
    for k in range(4):
        ends = plsc.load_gather(tab_v, [iota * L + (k * L * L + L - 1)])
        inc = plsc.cumsum(ends) + carry
        exc = inc - ends
        if k < 3:
            offs_v[pl.ds(k * L, L)] = exc
            tmp_v[...] = inc
            carry = plsc.load_gather(tmp_v, [zeros + (L - 1)])
        else:
            offs_v[pl.ds(k * L, L)] = jnp.where(iota == L - 1, 0.0, exc)

    cp_ts.wait()
    for i in range(BPW // L):
        t = ts_v[pl.ds(i * L, L)]
        idx = jnp.bitwise_and(t - 1, TAB_PAD - 1)
        out_v[pl.ds(i * L, L)] = (
            plsc.load_gather(tab_v, [idx])
            + plsc.load_gather(offs_v, [jnp.right_shift(idx, 4)]))

    pltpu.sync_copy(out_v, out_hbm.at[pl.ds(base, BPW)])


@jax.jit
def kernel(W, slope, power, w_ini, timesteps):
    del power
    pack = jnp.concatenate([
        W.astype(jnp.float32),
        jnp.reshape(w_ini.astype(jnp.float32), (1,)),
        slope.astype(jnp.float32),
        jnp.zeros((W_PACK - NUM_TIMESTEPS - 2,), jnp.float32),
    ])
    pack = jnp.reshape(jnp.tile(pack, (NW,)), (NW * W_PACK,))
    run = pl.kernel(
        _snr_body,
        out_type=jax.ShapeDtypeStruct((BATCH,), jnp.float32),
        mesh=plsc.VectorSubcoreMesh(core_axis_name="c", subcore_axis_name="s"),
        compiler_params=pltpu.CompilerParams(needs_layout_passes=False),
        scratch_types=[
            pltpu.VMEM((W_PACK,), jnp.float32),
            pltpu.VMEM((TAB_PAD,), jnp.float32),
            pltpu.VMEM((4 * L,), jnp.float32),
            pltpu.VMEM((L,), jnp.float32),
            pltpu.VMEM((BPW,), jnp.int32),
            pltpu.VMEM((BPW,), jnp.float32),
            pltpu.SemaphoreType.DMA,
            pltpu.SemaphoreType.DMA,
        ],
    )
    return run(pack, timesteps)


# speedup vs baseline: 1.0651x; 1.0238x over previous
import jax
import jax.numpy as jnp
from jax import lax
from jax.experimental import pallas as pl
from jax.experimental.pallas import tpu as pltpu
from jax.experimental.pallas import tpu_sc as plsc

NUM_TIMESTEPS = 1000
BATCH = 16384
L = 16
NC, NS = 2, 16
NW = NC * NS
BPW = BATCH // NW
NCHUNK = 63
W_PACK = NCHUNK * L
TAB_PAD = 1024


def _snr_body(pack_hbm, ts_hbm, out_hbm,
              w_v, tab_v, offs_v, tmp_v, ts_v, out_v, sem_ts, sem_w):
    wid = lax.axis_index("s") * NC + lax.axis_index("c")
    base = wid * BPW

    cp_w = pltpu.async_copy(pack_hbm.at[pl.ds(wid * W_PACK, W_PACK)], w_v,
                            sem_w)
    cp_ts = pltpu.async_copy(ts_hbm.at[pl.ds(base, BPW)], ts_v, sem_ts)

    neg_inf = jnp.full((L,), -jnp.inf, jnp.float32)
    tab_v[pl.ds(TAB_PAD - L, L)] = neg_inf

    cp_w.wait()
    zeros = jnp.zeros((L,), jnp.int32)
    wini_vec = plsc.load_gather(w_v, [zeros + NUM_TIMESTEPS])
    slope_vec = plsc.load_gather(w_v, [zeros + (NUM_TIMESTEPS + 1)])

    @plsc.parallel_loop(0, NCHUNK, unroll=7)
    def _build(j):
        v = jnp.maximum(w_v[pl.ds(j * L, L)] + wini_vec, 0.0)
        tab_v[pl.ds(j * L, L)] = plsc.cumsum(v)

    iota = lax.iota(jnp.int32, L)
    carry = 0.0 - slope_vec
    for k in range(4):
        ends = plsc.load_gather(tab_v, [iota * L + (k * L * L + L - 1)])
        inc = plsc.cumsum(ends) + carry
        exc = inc - ends
        if k < 3:
            offs_v[pl.ds(k * L, L)] = exc
            tmp_v[...] = inc
            carry = plsc.load_gather(tmp_v, [zeros + (L - 1)])
        else:
            offs_v[pl.ds(k * L, L)] = jnp.where(iota == L - 1, 0.0, exc)

    cp_ts.wait()

    @plsc.parallel_loop(0, BPW // L, unroll=8)
    def _gather(i):
        t = ts_v[pl.ds(i * L, L)]
        idx = jnp.bitwise_and(t - 1, TAB_PAD - 1)
        out_v[pl.ds(i * L, L)] = (
            plsc.load_gather(tab_v, [idx])
            + plsc.load_gather(offs_v, [jnp.right_shift(idx, 4)]))

    pltpu.sync_copy(out_v, out_hbm.at[pl.ds(base, BPW)])


@jax.jit
def kernel(W, slope, power, w_ini, timesteps):
    del power
    pack = jnp.concatenate([
        W.astype(jnp.float32),
        jnp.reshape(w_ini.astype(jnp.float32), (1,)),
        slope.astype(jnp.float32),
        jnp.zeros((W_PACK - NUM_TIMESTEPS - 2,), jnp.float32),
    ])
    pack = jnp.reshape(jnp.tile(pack, (NW,)), (NW * W_PACK,))
    run = pl.kernel(
        _snr_body,
        out_type=jax.ShapeDtypeStruct((BATCH,), jnp.float32),
        mesh=plsc.VectorSubcoreMesh(core_axis_name="c", subcore_axis_name="s"),
        compiler_params=pltpu.CompilerParams(needs_layout_passes=False),
        scratch_types=[
            pltpu.VMEM((W_PACK,), jnp.float32),
            pltpu.VMEM((TAB_PAD,), jnp.float32),
            pltpu.VMEM((4 * L,), jnp.float32),
            pltpu.VMEM((L,), jnp.float32),
            pltpu.VMEM((BPW,), jnp.int32),
            pltpu.VMEM((BPW,), jnp.float32),
            pltpu.SemaphoreType.DMA,
            pltpu.SemaphoreType.DMA,
        ],
    )
    return run(pack, timesteps)


# speedup vs baseline: 1.0733x; 1.0077x over previous
import jax
import jax.numpy as jnp
from jax import lax
from jax.experimental import pallas as pl
from jax.experimental.pallas import tpu as pltpu
from jax.experimental.pallas import tpu_sc as plsc

NUM_TIMESTEPS = 1000
BATCH = 16384
L = 16
NC, NS = 2, 16
NW = NC * NS
BPW = BATCH // NW
NCHUNK = 63
W_PACK = NCHUNK * L
TAB_PAD = 1024


def _snr_body(pack_hbm, ts_hbm, out_hbm,
              w_v, tab_v, offs_v, tmp_v, ts_v, out_v, sem_ts, sem_w):
    wid = lax.axis_index("s") * NC + lax.axis_index("c")
    base = wid * BPW

    cp_w = pltpu.async_copy(pack_hbm.at[pl.ds(wid * W_PACK, W_PACK)], w_v,
                            sem_w)
    cp_ts = pltpu.async_copy(ts_hbm.at[pl.ds(base, BPW)], ts_v, sem_ts)

    neg_inf = jnp.full((L,), -jnp.inf, jnp.float32)
    tab_v[pl.ds(TAB_PAD - L, L)] = neg_inf

    cp_w.wait()
    zeros = jnp.zeros((L,), jnp.int32)
    wini_vec = plsc.load_gather(w_v, [zeros + NUM_TIMESTEPS])
    slope_vec = plsc.load_gather(w_v, [zeros + (NUM_TIMESTEPS + 1)])

    @plsc.parallel_loop(0, NCHUNK, unroll=7)
    def _build(j):
        v = jnp.maximum(w_v[pl.ds(j * L, L)] + wini_vec, 0.0)
        tab_v[pl.ds(j * L, L)] = plsc.cumsum(v)

    iota = lax.iota(jnp.int32, L)
    carry = 0.0 - slope_vec
    for k in range(4):
        ends = plsc.load_gather(tab_v, [iota * L + (k * L * L + L - 1)])
        inc = plsc.cumsum(ends) + carry
        exc = inc - ends
        if k < 3:
            offs_v[pl.ds(k * L, L)] = exc
            tmp_v[...] = inc
            carry = plsc.load_gather(tmp_v, [zeros + (L - 1)])
        else:
            offs_v[pl.ds(k * L, L)] = jnp.where(iota == L - 1, 0.0, exc)

    cp_ts.wait()
    half = BPW // 2

    @plsc.parallel_loop(0, BPW // L // 2, unroll=8)
    def _gather_lo(i):
        t = ts_v[pl.ds(i * L, L)]
        idx = jnp.bitwise_and(t - 1, TAB_PAD - 1)
        out_v[pl.ds(i * L, L)] = (
            plsc.load_gather(tab_v, [idx])
            + plsc.load_gather(offs_v, [jnp.right_shift(idx, 4)]))

    cp_lo = pltpu.async_copy(out_v.at[pl.ds(0, half)],
                             out_hbm.at[pl.ds(base, half)], sem_ts)

    @plsc.parallel_loop(BPW // L // 2, BPW // L, unroll=8)
    def _gather_hi(i):
        t = ts_v[pl.ds(i * L, L)]
        idx = jnp.bitwise_and(t - 1, TAB_PAD - 1)
        out_v[pl.ds(i * L, L)] = (
            plsc.load_gather(tab_v, [idx])
            + plsc.load_gather(offs_v, [jnp.right_shift(idx, 4)]))

    cp_hi = pltpu.async_copy(out_v.at[pl.ds(half, half)],
                             out_hbm.at[pl.ds(base + half, half)], sem_w)
    cp_lo.wait()
    cp_hi.wait()


@jax.jit
def kernel(W, slope, power, w_ini, timesteps):
    del power
    pack = jnp.concatenate([
        W.astype(jnp.float32),
        jnp.reshape(w_ini.astype(jnp.float32), (1,)),
        slope.astype(jnp.float32),
        jnp.zeros((W_PACK - NUM_TIMESTEPS - 2,), jnp.float32),
    ])
    pack = jnp.reshape(jnp.tile(pack, (NW,)), (NW * W_PACK,))
    run = pl.kernel(
        _snr_body,
        out_type=jax.ShapeDtypeStruct((BATCH,), jnp.float32),
        mesh=plsc.VectorSubcoreMesh(core_axis_name="c", subcore_axis_name="s"),
        compiler_params=pltpu.CompilerParams(needs_layout_passes=False),
        scratch_types=[
            pltpu.VMEM((W_PACK,), jnp.float32),
            pltpu.VMEM((TAB_PAD,), jnp.float32),
            pltpu.VMEM((4 * L,), jnp.float32),
            pltpu.VMEM((L,), jnp.float32),
            pltpu.VMEM((BPW,), jnp.int32),
            pltpu.VMEM((BPW,), jnp.float32),
            pltpu.SemaphoreType.DMA,
            pltpu.SemaphoreType.DMA,
        ],
    )
    return run(pack, timesteps)
